# Initial kernel scaffold; baseline (speedup 1.0000x reference)
import jax
import jax.numpy as jnp
from jax.experimental import pallas as pl


def kernel(GLA_img_aug, lbl, index, param, conv_w1, conv_b1, conv_w2, conv_b2):
    raise NotImplementedError("write your pallas kernel here")


# speedup vs baseline: 24.0471x; 24.0471x over previous
import functools

import jax
import jax.numpy as jnp
from jax import lax
from jax.experimental import pallas as pl
from jax.experimental.pallas import tpu as pltpu
from jax.experimental.pallas import tpu_sc as plsc

CNUM = 5
AUG = 4
LNUM = 3
H = 224
W = 224

R0 = 8
C0 = 128
PR = 240
PC = 384


def _sc_gather(index, w12d, b12d, w22d):
    B = index.shape[0]
    tables = (w12d, b12d, w22d)
    dims = tuple(t.shape[1] for t in tables)
    mesh = plsc.VectorSubcoreMesh(core_axis_name="c", subcore_axis_name="s")

    @functools.partial(
        pl.kernel,
        mesh=mesh,
        compiler_params=pltpu.CompilerParams(use_tc_tiling_on_sc=False),
        out_type=[jax.ShapeDtypeStruct((B, d), jnp.float32) for d in dims],
        scratch_types=[pltpu.VMEM((B,), jnp.int32)]
        + [pltpu.VMEM((B, d), jnp.float32) for d in dims]
        + [pltpu.SemaphoreType.DMA],
    )
    def gather_k(idx_hbm, t0, t1, t2, o0, o1, o2, idx_v, b0, b1, b2, sem):
        wid = lax.axis_index("s") * 2 + lax.axis_index("c")
        ts = (t0, t1, t2)
        os_ = (o0, o1, o2)
        bs = (b0, b1, b2)
        for t in range(3):
            @pl.when(wid == t)
            def _(t=t):
                pltpu.sync_copy(idx_hbm, idx_v)
                pltpu.async_copy(ts[t].at[idx_v], bs[t], sem).wait()
                pltpu.sync_copy(bs[t], os_[t])

    return gather_k(index, *tables)


def _sigm(x):
    return 1.0 / (1.0 + jnp.exp(-x))


def _tc_body(idx_ref, img_ref, lbl_ref, p_ref, b2t_ref, w1_ref, b1_ref,
             w2_ref, out_ref, pimg, ph, csc, stats):
    b = pl.program_id(0)
    i = pl.program_id(1)
    l = pl.program_id(2)

    @pl.when(jnp.logical_and(b == 0, jnp.logical_and(i == 0, l == 0)))
    def _zero_halo():
        pimg[...] = jnp.zeros((PR, PC), jnp.float32)
        ph[...] = jnp.zeros((4, PR, PC), jnp.float32)

    @pl.when(jnp.logical_and(i == 0, l == 0))
    def _init_sample():
        x = img_ref[0]
        mean = jnp.mean(x)
        var = jnp.mean((x - mean) * (x - mean))
        std = jnp.sqrt(var) + 1e-6
        stats[0] = mean
        stats[1] = std
        pimg[R0:R0 + H, C0:C0 + W] = (x - mean) / std

    mask = lbl_ref[0] == i

    @pl.when(l == 0)
    def _seed_c():
        csc[...] = jnp.where(mask, pimg[R0:R0 + H, C0:C0 + W], 0.0)

    taps = [
        pimg[R0 + ky - 1:R0 + ky - 1 + H, C0 + kx - 1:C0 + kx - 1 + W]
        for ky in range(3) for kx in range(3)
    ]
    for k in range(4):
        acc = jnp.full((H, W), b1_ref[0, i, l, k], jnp.float32)
        for t in range(9):
            acc = acc + w1_ref[0, i, l, k * 9 + t] * taps[t]
        ph[k, R0:R0 + H, C0:C0 + W] = acc

    o = jnp.full((H, W), b2t_ref[0, 0, i * (AUG * LNUM) + l], jnp.float32)
    for k in range(4):
        for ky in range(3):
            for kx in range(3):
                o = o + w2_ref[0, i, l, k * 9 + ky * 3 + kx] * ph[
                    k, R0 + ky - 1:R0 + ky - 1 + H, C0 + kx - 1:C0 + kx - 1 + W]
    mix = _sigm(o)

    pbase = i * (AUG * LNUM * 7) + l * 7
    p1 = _sigm(p_ref[0, 0, pbase + 0])
    p2 = _sigm(p_ref[0, 0, pbase + 1])
    p1v = _sigm(p_ref[0, 0, pbase + 2])
    p2v = _sigm(p_ref[0, 0, pbase + 3])
    c = csc[...]
    u = 1.0 - c
    uuc3 = 3.0 * u * u * c
    ucc3 = 3.0 * u * c * c
    ct = uuc3 * p1 + ucc3 * p2 + c * c * c
    cv = u * u * u + uuc3 * p1v + ucc3 * p2v
    cnew = jnp.clip(ct * mix + cv * (1.0 - mix), 0.0, 1.0)
    csc[...] = cnew

    @pl.when(l == LNUM - 1)
    def _merge():
        img = pimg[R0:R0 + H, C0:C0 + W]
        pimg[R0:R0 + H, C0:C0 + W] = jnp.where(mask, cnew, img)

    @pl.when(jnp.logical_and(i == CNUM - 1, l == LNUM - 1))
    def _emit():
        out_ref[0] = pimg[R0:R0 + H, C0:C0 + W] * stats[1] + stats[0]


def _tc_pipeline(index, img, lbl, p2d, b22d, w1, b1, w2):
    B = img.shape[0]
    grid = (B, CNUM, LNUM)
    smem = functools.partial(pl.BlockSpec, memory_space=pltpu.SMEM)
    grid_spec = pltpu.PrefetchScalarGridSpec(
        num_scalar_prefetch=1,
        grid=grid,
        in_specs=[
            pl.BlockSpec((1, H, W), lambda b, i, l, idx: (b, 0, 0)),
            pl.BlockSpec((1, H, W), lambda b, i, l, idx: (b, 0, 0)),
            smem((1, 1, 420), lambda b, i, l, idx: (idx[b], 0, 0)),
            smem((1, 1, 60), lambda b, i, l, idx: (idx[b], 0, 0)),
            smem((1, CNUM, LNUM, 36), lambda b, i, l, idx: (b, 0, 0, 0)),
            smem((1, CNUM, LNUM, 4), lambda b, i, l, idx: (b, 0, 0, 0)),
            smem((1, CNUM, LNUM, 36), lambda b, i, l, idx: (b, 0, 0, 0)),
        ],
        out_specs=pl.BlockSpec((1, H, W), lambda b, i, l, idx: (b, 0, 0)),
        scratch_shapes=[
            pltpu.VMEM((PR, PC), jnp.float32),
            pltpu.VMEM((4, PR, PC), jnp.float32),
            pltpu.VMEM((H, W), jnp.float32),
            pltpu.SMEM((2,), jnp.float32),
        ],
    )
    return pl.pallas_call(
        _tc_body,
        grid_spec=grid_spec,
        out_shape=jax.ShapeDtypeStruct((B, H, W), jnp.float32),
    )(index, img, lbl, p2d, b22d, w1, b1, w2)


def kernel(GLA_img_aug, lbl, index, param, conv_w1, conv_b1, conv_w2, conv_b2):
    B = GLA_img_aug.shape[0]
    D = param.shape[0]
    idx = index.astype(jnp.int32)
    w1_rows, b1_rows, w2_rows = _sc_gather(
        idx,
        conv_w1.reshape(D, -1),
        conv_b1.reshape(D, -1),
        conv_w2.reshape(D, -1),
    )
    w1 = w1_rows.reshape(B, CNUM, AUG, LNUM, 36)[:, :, 0]
    b1 = b1_rows.reshape(B, CNUM, AUG, LNUM, 4)[:, :, 0]
    w2 = w2_rows.reshape(B, CNUM, AUG, LNUM, 36)[:, :, 0]
    out = _tc_pipeline(
        idx, GLA_img_aug.reshape(B, H, W), lbl,
        param.reshape(D, 1, -1), conv_b2.reshape(D, 1, -1), w1, b1, w2)
    return out.reshape(B, 1, H, W)
